# baseline (speedup 1.0000x reference)
import functools

import jax
import jax.numpy as jnp
from jax import lax
from jax.experimental import pallas as pl
from jax.experimental.pallas import tpu as pltpu
from jax.experimental.pallas import tpu_sc as plsc

_C1 = 96
_CE = 96
_CAND = _C1 * (_C1 - 1) // 2
_NS = _CAND // 16
_NG = (_NS + 15) // 16
_HW = 56 * 56
_B = 32
_N = float(_B * _HW)
_CP = 128


_CCH = 8
_NCH = _C1 // _CCH


def _sc_topk_body(logits_hbm, gumbel_hbm, cm0_hbm, cm1_hbm, x_hbm,
                  sel_hbm, outf_hbm,
                  lg_v, gu_v, scores_v, idx_v, rows_v, chunk_v, sem,
                  rsems, wsems):
    cid = lax.axis_index("c")
    sid = lax.axis_index("s")
    wid = sid * 2 + cid

    @pl.when(wid > 0)
    def _copy():
        base = (wid - 1) * 13

        def _rd(h, s):
            b = lax.div(h, _NCH)
            c = lax.rem(h, _NCH)
            return pltpu.make_async_copy(
                x_hbm.at[b, pl.ds(c * _CCH, _CCH)], chunk_v.at[s],
                rsems.at[s])

        def _wr(h, s):
            b = lax.div(h, _NCH)
            c = lax.rem(h, _NCH)
            return pltpu.make_async_copy(
                chunk_v.at[s], outf_hbm.at[b, pl.ds(c * _CCH, _CCH)],
                wsems.at[s])

        n_h = jnp.minimum(jnp.int32(13), jnp.int32(_B * _NCH) - base)

        @pl.when(n_h > 0)
        def _():
            _rd(base, 0).start()

            def copy_loop(j, carry):
                s = lax.rem(j, 2)

                @pl.when(j + 1 < n_h)
                def _():
                    @pl.when(j >= 1)
                    def _():
                        _wr(0, 1 - s).wait()

                    _rd(base + j + 1, 1 - s).start()

                _rd(base + j, s).wait()
                _wr(base + j, s).start()
                return carry

            lax.fori_loop(0, n_h, copy_loop, 0)

            @pl.when(n_h >= 2)
            def _():
                _wr(0, lax.rem(n_h - 2, 2)).wait()

            _wr(0, lax.rem(n_h - 1, 2)).wait()

    @pl.when((cid == 0) & (sid == 0))
    def _():
        pltpu.sync_copy(logits_hbm, lg_v)
        pltpu.sync_copy(gumbel_hbm, gu_v)
        iota = lax.iota(jnp.int32, 16)
        lane0 = iota == 0
        neg_inf = jnp.float32(-jnp.inf)

        def fill(t, carry):
            ix = t * 16 + iota
            sc = plsc.load_gather(lg_v, [ix]) + plsc.load_gather(gu_v, [ix])
            plsc.store_scatter(scores_v, [ix], sc)
            return carry

        lax.fori_loop(0, _NS, fill, 0)
        for t in range(_NS, _NG * 16):
            plsc.store_scatter(scores_v, [t * 16 + iota],
                               jnp.full((16,), neg_inf, jnp.float32))

        maxima = []
        for g in range(_NG):
            m_g = jnp.full((16,), neg_inf, jnp.float32)
            for j in range(16):
                v = plsc.load_gather(scores_v, [256 * g + 16 * iota + j])
                m_g = jnp.maximum(m_g, v)
            maxima.append(m_g)

        def extract(k, maxima):
            bv = maxima[0]
            bg = jnp.zeros((16,), jnp.int32)
            for g in range(1, _NG):
                take = maxima[g] > bv
                bv = jnp.where(take, maxima[g], bv)
                bg = jnp.where(take, jnp.int32(g), bg)
            m = jnp.max(bv)
            s_star = jnp.min(jnp.where(bv == m, bg * 16 + iota,
                                       jnp.int32(1 << 30)))
            six = s_star * 16 + iota
            sl = plsc.load_gather(scores_v, [six])
            c_star = jnp.min(jnp.where(sl == m, six, jnp.int32(1 << 30)))
            plsc.store_scatter(idx_v, [jnp.full((16,), k, jnp.int32)],
                               jnp.full((16,), c_star, jnp.int32), mask=lane0)
            plsc.store_scatter(scores_v, [jnp.full((16,), c_star, jnp.int32)],
                               jnp.full((16,), neg_inf, jnp.float32),
                               mask=lane0)
            m2 = jnp.max(jnp.where(six == c_star, neg_inf, sl))
            gs = lax.div(s_star, 16)
            ls = lax.rem(s_star, 16)
            upd = iota == ls
            out = []
            for g in range(_NG):
                hit = (jnp.full((16,), gs, jnp.int32) == g) & upd
                out.append(jnp.where(hit, jnp.full((16,), m2, jnp.float32),
                                     maxima[g]))
            return tuple(out)

        lax.fori_loop(0, _CE, extract, tuple(maxima))

        pltpu.async_copy(cm0_hbm.at[idx_v], rows_v, sem).wait()
        pltpu.sync_copy(rows_v, sel_hbm.at[0])
        pltpu.async_copy(cm1_hbm.at[idx_v], rows_v, sem).wait()
        pltpu.sync_copy(rows_v, sel_hbm.at[1])


def _sc_topk(logits, gumbel, cm0, cm1, xf):
    mesh = plsc.VectorSubcoreMesh(core_axis_name="c", subcore_axis_name="s")
    fn = pl.kernel(
        _sc_topk_body,
        out_type=[jax.ShapeDtypeStruct((2, _CE, _CP), jnp.float32),
                  jax.ShapeDtypeStruct((_B, _C1 + _CE, _HW), jnp.float32)],
        scratch_types=[
            pltpu.VMEM((_CAND,), jnp.float32),
            pltpu.VMEM((_CAND,), jnp.float32),
            pltpu.VMEM((_NG * 256,), jnp.float32),
            pltpu.VMEM((_CE,), jnp.int32),
            pltpu.VMEM((_CE, _CP), jnp.float32),
            pltpu.VMEM((2, _CCH, _HW), jnp.float32),
            pltpu.SemaphoreType.DMA,
            pltpu.SemaphoreType.DMA((2,)),
            pltpu.SemaphoreType.DMA((2,)),
        ],
        mesh=mesh,
        compiler_params=pltpu.CompilerParams(needs_layout_passes=False),
    )
    return fn(logits, gumbel, cm0, cm1, xf)


_NB = 4
_NO = 3
_DEPTH = 3


def _in_cp(x_hbm, xbuf, in_sems, b, slot):
    return pltpu.make_async_copy(x_hbm.at[b], xbuf.at[slot], in_sems.at[slot])


def _eout_cp(out_hbm, obuf, ob_sems, b, slot):
    return pltpu.make_async_copy(obuf.at[slot], out_hbm.at[b, pl.ds(_C1, _CE)],
                                 ob_sems.at[slot])


def _tc_body(x_hbm, sel_ref, gamma_ref, beta_ref, out0_hbm, out_hbm,
             xbuf, obuf, p_scr, sum_ref, sq_ref, scale_ref, shift_ref,
             in_sems, ob_sems):
    del out0_hbm
    sum_ref[...] = jnp.zeros_like(sum_ref)
    sq_ref[...] = jnp.zeros_like(sq_ref)

    for b in range(_DEPTH):
        _in_cp(x_hbm, xbuf, in_sems, b, b).start()

    def ph1(b, carry):
        slot = lax.rem(b, _NB)
        _in_cp(x_hbm, xbuf, in_sems, b, slot).wait()

        @pl.when(b + _DEPTH < _B)
        def _():
            _in_cp(x_hbm, xbuf, in_sems, b + _DEPTH,
                   lax.rem(b + _DEPTH, _NB)).start()

        xb = xbuf[slot]
        xi = jnp.dot(sel_ref[0][:, :_C1], xb,
                     preferred_element_type=jnp.float32)
        xj = jnp.dot(sel_ref[1][:, :_C1], xb,
                     preferred_element_type=jnp.float32)
        p = xi * xj
        p_scr[b] = p
        sum_ref[...] += jnp.sum(p, axis=1, keepdims=True)
        sq_ref[...] += jnp.sum(p * p, axis=1, keepdims=True)
        return carry

    lax.fori_loop(0, _B, ph1, 0)

    mean = sum_ref[...] / _N
    var = sq_ref[...] / _N - mean * mean
    scale = gamma_ref[...] * lax.rsqrt(var + 1e-5)
    scale_ref[...] = scale
    shift_ref[...] = beta_ref[...] - mean * scale

    def ph2(b, carry):
        slot = lax.rem(b, _NO)

        @pl.when(b >= _NO)
        def _():
            _eout_cp(out_hbm, obuf, ob_sems, 0, slot).wait()

        obuf[slot] = p_scr[b] * scale_ref[...] + shift_ref[...]
        _eout_cp(out_hbm, obuf, ob_sems, b, slot).start()
        return carry

    lax.fori_loop(0, _B, ph2, 0)

    for b in range(_B - _NO, _B):
        _eout_cp(out_hbm, obuf, ob_sems, 0, b % _NO).wait()


def _tc_expand(xf, sel, gamma2, beta2, out0):
    return pl.pallas_call(
        _tc_body,
        in_specs=[
            pl.BlockSpec(memory_space=pltpu.MemorySpace.HBM),
            pl.BlockSpec(memory_space=pltpu.MemorySpace.VMEM),
            pl.BlockSpec(memory_space=pltpu.MemorySpace.VMEM),
            pl.BlockSpec(memory_space=pltpu.MemorySpace.VMEM),
            pl.BlockSpec(memory_space=pltpu.MemorySpace.HBM),
        ],
        out_specs=pl.BlockSpec(memory_space=pltpu.MemorySpace.HBM),
        out_shape=jax.ShapeDtypeStruct((_B, _C1 + _CE, _HW), jnp.float32),
        input_output_aliases={4: 0},
        scratch_shapes=[
            pltpu.VMEM((_NB, _C1, _HW), jnp.float32),
            pltpu.VMEM((_NO, _CE, _HW), jnp.float32),
            pltpu.VMEM((_B, _CE, _HW), jnp.float32),
            pltpu.VMEM((_CE, 1), jnp.float32),
            pltpu.VMEM((_CE, 1), jnp.float32),
            pltpu.VMEM((_CE, 1), jnp.float32),
            pltpu.VMEM((_CE, 1), jnp.float32),
            pltpu.SemaphoreType.DMA((_NB,)),
            pltpu.SemaphoreType.DMA((_NO,)),
        ],
    )(xf, sel, gamma2, beta2, out0)


@jax.jit
def kernel(x, logits, tau, gamma, beta, gumbel, candis_met):
    del tau
    cmp_pad = jnp.pad(candis_met, ((0, 0), (0, 0), (0, _CP - _C1)))
    B, C, H, W = x.shape
    xf = x.reshape(B, C, H * W)
    sel, out0 = _sc_topk(logits, gumbel, cmp_pad[0], cmp_pad[1], xf)
    out = _tc_expand(xf, sel, gamma.reshape(_CE, 1), beta.reshape(_CE, 1), out0)
    return out.reshape(B, 2 * C, H, W)


# speedup vs baseline: 1.1398x; 1.1398x over previous
import functools

import jax
import jax.numpy as jnp
from jax import lax
from jax.experimental import pallas as pl
from jax.experimental.pallas import tpu as pltpu
from jax.experimental.pallas import tpu_sc as plsc

_C1 = 96
_CE = 96
_CAND = _C1 * (_C1 - 1) // 2
_NS = _CAND // 16
_NG = (_NS + 15) // 16
_HW = 56 * 56
_B = 32
_N = float(_B * _HW)
_CP = 128


def _sc_topk_body(logits_hbm, gumbel_hbm, out_hbm,
                  lg_v, gu_v, scores_v, idx_v, oh0_v, oh1_v, sem):
    cid = lax.axis_index("c")
    sid = lax.axis_index("s")

    @pl.when((cid == 0) & (sid == 0))
    def _():
        pltpu.sync_copy(logits_hbm, lg_v)
        pltpu.sync_copy(gumbel_hbm, gu_v)
        iota = lax.iota(jnp.int32, 16)
        lane0 = iota == 0
        neg_inf = jnp.float32(-jnp.inf)

        def fill(t, carry):
            ix = t * 16 + iota
            sc = plsc.load_gather(lg_v, [ix]) + plsc.load_gather(gu_v, [ix])
            plsc.store_scatter(scores_v, [ix], sc)
            return carry

        lax.fori_loop(0, _NS, fill, 0)
        for t in range(_NS, _NG * 16):
            plsc.store_scatter(scores_v, [t * 16 + iota],
                               jnp.full((16,), neg_inf, jnp.float32))

        maxima = []
        for g in range(_NG):
            m_g = jnp.full((16,), neg_inf, jnp.float32)
            for j in range(16):
                v = plsc.load_gather(scores_v, [256 * g + 16 * iota + j])
                m_g = jnp.maximum(m_g, v)
            maxima.append(m_g)

        def extract(k, maxima):
            bv = maxima[0]
            bg = jnp.zeros((16,), jnp.int32)
            for g in range(1, _NG):
                take = maxima[g] > bv
                bv = jnp.where(take, maxima[g], bv)
                bg = jnp.where(take, jnp.int32(g), bg)
            m = jnp.max(bv)
            s_star = jnp.min(jnp.where(bv == m, bg * 16 + iota,
                                       jnp.int32(1 << 30)))
            six = s_star * 16 + iota
            sl = plsc.load_gather(scores_v, [six])
            c_star = jnp.min(jnp.where(sl == m, six, jnp.int32(1 << 30)))
            plsc.store_scatter(idx_v, [jnp.full((16,), k, jnp.int32)],
                               jnp.full((16,), c_star, jnp.int32), mask=lane0)
            plsc.store_scatter(scores_v, [jnp.full((16,), c_star, jnp.int32)],
                               jnp.full((16,), neg_inf, jnp.float32),
                               mask=lane0)
            m2 = jnp.max(jnp.where(six == c_star, neg_inf, sl))
            gs = lax.div(s_star, 16)
            ls = lax.rem(s_star, 16)
            upd = iota == ls
            out = []
            for g in range(_NG):
                hit = (jnp.full((16,), gs, jnp.int32) == g) & upd
                out.append(jnp.where(hit, jnp.full((16,), m2, jnp.float32),
                                     maxima[g]))
            return tuple(out)

        lax.fori_loop(0, _CE, extract, tuple(maxima))

        def zero(t, carry):
            ix = t * 16 + iota
            z = jnp.zeros((16,), jnp.float32)
            plsc.store_scatter(oh0_v, [ix], z)
            plsc.store_scatter(oh1_v, [ix], z)
            return carry

        lax.fori_loop(0, _CE * _CP // 16, zero, 0)

        ones_v = jnp.full((16,), 1.0, jnp.float32)
        for t in range(_CE // 16):
            c = plsc.load_gather(idx_v, [t * 16 + iota])
            i = jnp.zeros((16,), jnp.int32)
            for ip in range(1, _C1):
                offi = 95 * ip - ip * (ip - 1) // 2
                i = i + jnp.where(c >= offi, jnp.int32(1), jnp.int32(0))
            offv = 95 * i - (i * (i - 1)) // 2
            j = c - offv + i + 1
            kbase = (t * 16 + iota) * _CP
            plsc.store_scatter(oh0_v, [kbase + i], ones_v)
            plsc.store_scatter(oh1_v, [kbase + j], ones_v)
        pltpu.sync_copy(oh0_v, out_hbm.at[0])
        pltpu.sync_copy(oh1_v, out_hbm.at[1])


def _sc_topk(logits, gumbel):
    mesh = plsc.VectorSubcoreMesh(core_axis_name="c", subcore_axis_name="s")
    fn = pl.kernel(
        _sc_topk_body,
        out_type=jax.ShapeDtypeStruct((2, _CE * _CP), jnp.float32),
        scratch_types=[
            pltpu.VMEM((_CAND,), jnp.float32),
            pltpu.VMEM((_CAND,), jnp.float32),
            pltpu.VMEM((_NG * 256,), jnp.float32),
            pltpu.VMEM((_CE,), jnp.int32),
            pltpu.VMEM((_CE * _CP,), jnp.float32),
            pltpu.VMEM((_CE * _CP,), jnp.float32),
            pltpu.SemaphoreType.DMA,
        ],
        mesh=mesh,
        compiler_params=pltpu.CompilerParams(needs_layout_passes=False),
    )
    return fn(logits, gumbel)


_NB = 6
_NO = 3
_DEPTH = 3


def _in_cp(x_hbm, xbuf, in_sems, b, slot):
    return pltpu.make_async_copy(x_hbm.at[b], xbuf.at[slot], in_sems.at[slot])


def _xout_cp(out_hbm, xbuf, cp_sems, b, slot):
    return pltpu.make_async_copy(xbuf.at[slot], out_hbm.at[b, pl.ds(0, _C1)],
                                 cp_sems.at[slot])


def _eout_cp(out_hbm, obuf, ob_sems, b, slot):
    return pltpu.make_async_copy(obuf.at[slot], out_hbm.at[b, pl.ds(_C1, _CE)],
                                 ob_sems.at[slot])


def _tc_body(x_hbm, sel_ref, gamma_ref, beta_ref, out_hbm,
             xbuf, obuf, p_scr, sum_ref, sq_ref, scale_ref, shift_ref,
             in_sems, cp_sems, ob_sems):
    sum_ref[...] = jnp.zeros_like(sum_ref)
    sq_ref[...] = jnp.zeros_like(sq_ref)

    for b in range(_DEPTH):
        _in_cp(x_hbm, xbuf, in_sems, b, b).start()

    def ph1(b, carry):
        slot = lax.rem(b, _NB)
        _in_cp(x_hbm, xbuf, in_sems, b, slot).wait()
        _xout_cp(out_hbm, xbuf, cp_sems, b, slot).start()

        @pl.when(b + _DEPTH < _B)
        def _():
            slot2 = lax.rem(b + _DEPTH, _NB)

            @pl.when(b >= _NB - _DEPTH)
            def _():
                _xout_cp(out_hbm, xbuf, cp_sems, 0, slot2).wait()

            _in_cp(x_hbm, xbuf, in_sems, b + _DEPTH, slot2).start()

        xb = xbuf[slot]
        xi = jnp.dot(sel_ref[0][:, :_C1], xb,
                     preferred_element_type=jnp.float32)
        xj = jnp.dot(sel_ref[1][:, :_C1], xb,
                     preferred_element_type=jnp.float32)
        p = xi * xj
        p_scr[b] = p
        sum_ref[...] += jnp.sum(p, axis=1, keepdims=True)
        sq_ref[...] += jnp.sum(p * p, axis=1, keepdims=True)
        return carry

    lax.fori_loop(0, _B, ph1, 0)

    mean = sum_ref[...] / _N
    var = sq_ref[...] / _N - mean * mean
    scale = gamma_ref[...] * lax.rsqrt(var + 1e-5)
    scale_ref[...] = scale
    shift_ref[...] = beta_ref[...] - mean * scale

    def ph2(b, carry):
        slot = lax.rem(b, _NO)

        @pl.when(b >= _NO)
        def _():
            _eout_cp(out_hbm, obuf, ob_sems, 0, slot).wait()

        obuf[slot] = p_scr[b] * scale_ref[...] + shift_ref[...]
        _eout_cp(out_hbm, obuf, ob_sems, b, slot).start()
        return carry

    lax.fori_loop(0, _B, ph2, 0)

    for b in range(_B - _NB, _B):
        _xout_cp(out_hbm, xbuf, cp_sems, 0, b % _NB).wait()
    for b in range(_B - _NO, _B):
        _eout_cp(out_hbm, obuf, ob_sems, 0, b % _NO).wait()


def _tc_expand(xf, sel, gamma2, beta2):
    return pl.pallas_call(
        _tc_body,
        in_specs=[
            pl.BlockSpec(memory_space=pltpu.MemorySpace.HBM),
            pl.BlockSpec(memory_space=pltpu.MemorySpace.VMEM),
            pl.BlockSpec(memory_space=pltpu.MemorySpace.VMEM),
            pl.BlockSpec(memory_space=pltpu.MemorySpace.VMEM),
        ],
        out_specs=pl.BlockSpec(memory_space=pltpu.MemorySpace.HBM),
        out_shape=jax.ShapeDtypeStruct((_B, _C1 + _CE, _HW), jnp.float32),
        scratch_shapes=[
            pltpu.VMEM((_NB, _C1, _HW), jnp.float32),
            pltpu.VMEM((_NO, _CE, _HW), jnp.float32),
            pltpu.VMEM((_B, _CE, _HW), jnp.float32),
            pltpu.VMEM((_CE, 1), jnp.float32),
            pltpu.VMEM((_CE, 1), jnp.float32),
            pltpu.VMEM((_CE, 1), jnp.float32),
            pltpu.VMEM((_CE, 1), jnp.float32),
            pltpu.SemaphoreType.DMA((_NB,)),
            pltpu.SemaphoreType.DMA((_NB,)),
            pltpu.SemaphoreType.DMA((_NO,)),
        ],
    )(xf, sel, gamma2, beta2)


@jax.jit
def kernel(x, logits, tau, gamma, beta, gumbel, candis_met):
    del tau
    del candis_met
    sel = _sc_topk(logits, gumbel).reshape(2, _CE, _CP)
    B, C, H, W = x.shape
    xf = x.reshape(B, C, H * W)
    out = _tc_expand(xf, sel, gamma.reshape(_CE, 1), beta.reshape(_CE, 1))
    return out.reshape(B, 2 * C, H, W)
